# Initial kernel scaffold; baseline (speedup 1.0000x reference)
import jax
import jax.numpy as jnp
from jax.experimental import pallas as pl


def kernel(y, We1, be1, We2, be2, Wt1, bt1, Wt2, bt2, Wd1, bd1, Wd2, bd2, Q, R):
    raise NotImplementedError("write your pallas kernel here")


# speedup vs baseline: 1.1734x; 1.1734x over previous
import jax
import jax.numpy as jnp
from jax.experimental import pallas as pl
from jax.experimental.pallas import tpu as pltpu

_T = 524288
_OBS = 3
_STATE = 6
_HID = 32
_CHUNK = 2048
_U = 8


def _dot(v, w):
    return jax.lax.dot_general(
        v, w, (((1,), (0,)), ((), ())), preferred_element_type=jnp.float32
    )


def _nssm_kernel(
    y_ref, We1, be1, We2, be2, Wt1, bt1, Wt2, bt2, Wd1, bd1, Wd2, bd2, R_smem,
    loss_out, lat_ref, h1_carry, loss_scr, cy_scr, h1_scr, h2_scr, h3_scr,
):
    we1 = We1[...]
    be1v = be1[...]
    we2 = We2[...]
    be2v = be2[...]
    wt1 = Wt1[...]
    bt1v = bt1[...]
    wt2 = Wt2[...]
    bt2v = bt2[...]
    wd1 = Wd1[...]
    bd1v = bd1[...]
    wd2 = Wd2[...]
    bd2v = bd2[...]

    @pl.when(pl.program_id(0) == 0)
    def _init():
        h1_carry[...] = jnp.maximum(bt1v, 0.0)
        loss_scr[...] = jnp.zeros_like(loss_scr)

    A = _dot(wt2, wd1)
    B = _dot(wd2, we1)
    C = _dot(wt2, wt1)
    D = _dot(we2, wt1)
    a_row = _dot(bt2v, wd1) + bd1v
    d_row = _dot(bt2v + be2v, wt1) + bt1v

    cy_scr[...] = _dot(y_ref[...], we1) + (be1v - _dot(bd2v, we1))

    def body(k, h1):
        cy8 = cy_scr[pl.ds(k * _U, _U), :]
        r1, r2, r3 = [], [], []
        for j in range(_U):
            h2 = jnp.maximum(_dot(h1, A) + a_row, 0.0)
            h3 = jnp.maximum(cy8[j : j + 1, :] - _dot(h2, B), 0.0)
            h1n = jnp.maximum(_dot(h1, C) + _dot(h3, D) + d_row, 0.0)
            r1.append(h1)
            r2.append(h2)
            r3.append(h3)
            h1 = h1n
        h1_scr[pl.ds(k * _U, _U), :] = jnp.concatenate(r1, axis=0)
        h2_scr[pl.ds(k * _U, _U), :] = jnp.concatenate(r2, axis=0)
        h3_scr[pl.ds(k * _U, _U), :] = jnp.concatenate(r3, axis=0)
        return h1

    h1f = jax.lax.fori_loop(0, _CHUNK // _U, body, h1_carry[...])
    h1_carry[...] = h1f

    H1 = h1_scr[...]
    H2 = h2_scr[...]
    H3 = h3_scr[...]
    lat_ref[...] = _dot(H1, wt2) + _dot(H3, we2) + (bt2v + be2v)
    innov = y_ref[...] - (_dot(H2, wd2) + bd2v)

    eps = jnp.float32(1e-5)
    a = R_smem[0, 0] + eps
    b = R_smem[0, 1]
    c = R_smem[0, 2]
    d = R_smem[1, 0]
    e = R_smem[1, 1] + eps
    f = R_smem[1, 2]
    g = R_smem[2, 0]
    h = R_smem[2, 1]
    i = R_smem[2, 2] + eps
    det = a * (e * i - f * h) - b * (d * i - f * g) + c * (d * h - e * g)
    idet = 1.0 / det
    one = jnp.ones((1, 1), jnp.float32)
    r0 = jnp.concatenate(
        [(e * i - f * h) * idet * one, (c * h - b * i) * idet * one,
         (b * f - c * e) * idet * one], axis=1)
    r1_ = jnp.concatenate(
        [(f * g - d * i) * idet * one, (a * i - c * g) * idet * one,
         (c * d - a * f) * idet * one], axis=1)
    r2_ = jnp.concatenate(
        [(d * h - e * g) * idet * one, (b * g - a * h) * idet * one,
         (a * e - b * d) * idet * one], axis=1)
    sinv = jnp.concatenate([r0, r1_, r2_], axis=0)

    z = innov * _dot(innov, sinv)
    loss_scr[...] = loss_scr[...] + jnp.sum(z, axis=(0, 1), keepdims=True)
    loss_out[...] = loss_scr[...] * (1.0 / _T)


def kernel(y, We1, be1, We2, be2, Wt1, bt1, Wt2, bt2, Wd1, bd1, Wd2, bd2, Q, R):
    del Q
    f32 = jnp.float32
    args = (
        y.astype(f32),
        We1.astype(f32), be1.astype(f32).reshape(1, _HID),
        We2.astype(f32), be2.astype(f32).reshape(1, _STATE),
        Wt1.astype(f32), bt1.astype(f32).reshape(1, _HID),
        Wt2.astype(f32), bt2.astype(f32).reshape(1, _STATE),
        Wd1.astype(f32), bd1.astype(f32).reshape(1, _HID),
        Wd2.astype(f32), bd2.astype(f32).reshape(1, _OBS),
        R.astype(f32),
    )
    grid = (_T // _CHUNK,)

    def full(shape):
        return pl.BlockSpec(shape, lambda i: (0, 0))

    in_specs = [
        pl.BlockSpec((_CHUNK, _OBS), lambda i: (i, 0)),
        full((_OBS, _HID)), full((1, _HID)),
        full((_HID, _STATE)), full((1, _STATE)),
        full((_STATE, _HID)), full((1, _HID)),
        full((_HID, _STATE)), full((1, _STATE)),
        full((_STATE, _HID)), full((1, _HID)),
        full((_HID, _OBS)), full((1, _OBS)),
        pl.BlockSpec(memory_space=pltpu.SMEM),
    ]
    out_specs = [
        pl.BlockSpec((1, 1), lambda i: (0, 0)),
        pl.BlockSpec((_CHUNK, _STATE), lambda i: (i, 0)),
    ]
    loss2d, latents = pl.pallas_call(
        _nssm_kernel,
        grid=grid,
        in_specs=in_specs,
        out_specs=out_specs,
        out_shape=[
            jax.ShapeDtypeStruct((1, 1), f32),
            jax.ShapeDtypeStruct((_T, _STATE), f32),
        ],
        scratch_shapes=[
            pltpu.VMEM((1, _HID), f32),
            pltpu.VMEM((1, 1), f32),
            pltpu.VMEM((_CHUNK, _HID), f32),
            pltpu.VMEM((_CHUNK, _HID), f32),
            pltpu.VMEM((_CHUNK, _HID), f32),
            pltpu.VMEM((_CHUNK, _HID), f32),
        ],
        compiler_params=pltpu.CompilerParams(
            dimension_semantics=("arbitrary",),
        ),
    )(*args)
    return loss2d[0, 0], latents
